# baseline (speedup 1.0000x reference)
import functools

import jax
import jax.numpy as jnp
from jax import lax
from jax.experimental import pallas as pl
from jax.experimental.pallas import tpu as pltpu
from jax.experimental.pallas import tpu_sc as plsc

N = 10000
E = 320000
NC = 2
NS = 16
CH = 128
NCH = E // CH
RPB = 624
F = 128


def _mesh():
    return plsc.VectorSubcoreMesh(core_axis_name="c", subcore_axis_name="s")


def _degree_call(dst2d):
    @functools.partial(
        pl.kernel,
        mesh=_mesh(),
        out_type=jax.ShapeDtypeStruct((NC * NS, N), jnp.float32),
        compiler_params=pltpu.CompilerParams(needs_layout_passes=False),
        scratch_types=[
            pltpu.VMEM((N,), jnp.float32),
            pltpu.VMEM((2, CH), jnp.int32),
            pltpu.SemaphoreType.DMA((2,)),
        ],
    )
    def k(dst_hbm, out_hbm, hist_v, idx_v, isem):
        c = lax.axis_index("c")
        s = lax.axis_index("s")
        tid = s * NC + c
        z16 = jnp.zeros((16,), jnp.float32)

        def zero_body(i, carry):
            hist_v[pl.ds(i * 16, 16)] = z16
            return carry

        lax.fori_loop(0, N // 16, zero_body, 0)

        ones = jnp.ones((16,), jnp.float32)
        stride = NC * NS
        nloop = (NCH + stride - 1) // stride

        for b in range(2):
            @pl.when(b * stride + tid < NCH)
            def _(b=b):
                pltpu.async_copy(dst_hbm.at[b * stride + tid], idx_v.at[b],
                                 isem.at[b])

        def chunk_body(kk, carry):
            for b in range(2):
                j = (kk * 2 + b) * stride + tid

                @pl.when(j < NCH)
                def _(b=b, j=j):
                    pltpu.make_async_copy(dst_hbm.at[j], idx_v.at[b],
                                          isem.at[b]).wait()
                    for i in range(CH // 16):
                        idx = idx_v[b, pl.ds(i * 16, 16)]
                        plsc.addupdate_scatter(hist_v, [idx], ones)
                    jn = j + 2 * stride

                    @pl.when(jn < NCH)
                    def _():
                        pltpu.async_copy(dst_hbm.at[jn], idx_v.at[b],
                                         isem.at[b])

            return carry

        lax.fori_loop(0, (nloop + 1) // 2, chunk_body, 0)
        pltpu.sync_copy(hist_v, out_hbm.at[tid])

    return k(dst2d)


def _make_prop():
    R = 3

    @functools.partial(
        pl.kernel,
        mesh=_mesh(),
        out_type=jax.ShapeDtypeStruct((NC, N, F), jnp.float32),
        scratch_types=[
            pltpu.VMEM((R, 4, CH // 2), jnp.int32),
            pltpu.VMEM((R, CH, F), jnp.float32),
            pltpu.VMEM_SHARED((N, F), jnp.float32),
            pltpu.SemaphoreType.DMA((R,)),
            pltpu.SemaphoreType.DMA((R,)),
            pltpu.SemaphoreType.DMA,
        ],
    )
    def k(ei_hbm, g_hbm, out_hbm, idx_v, rows_v, slab, gsem, ssem, isem):
        c = lax.axis_index("c")
        s = lax.axis_index("s")
        z16 = jnp.zeros((16,), jnp.float32)

        def zb(i, carry):
            rows_v[0, i // (F // 16), pl.ds((i % (F // 16)) * 16, 16)] = z16
            return carry

        lax.fori_loop(0, CH * (F // 16), zb, 0)
        for q in range(4):
            pltpu.sync_copy(rows_v.at[0],
                            slab.at[pl.ds(s * RPB + q * CH, CH)])
        pltpu.sync_copy(rows_v.at[0, pl.ds(0, RPB - 4 * CH)],
                        slab.at[pl.ds(s * RPB + 4 * CH, RPB - 4 * CH)])

        @pl.when(s == 0)
        def _zero_tail():
            pltpu.sync_copy(rows_v.at[0, pl.ds(0, N - NS * RPB)],
                            slab.at[pl.ds(NS * RPB, N - NS * RPB)])

        plsc.subcore_barrier()

        stride = NC * NS
        base = s * NC + c
        nloop = (NCH + stride - 1) // stride

        def body(kk, carry):
            js = [(kk * R + b) * stride + base for b in range(R)]

            for b in range(R):
                @pl.when(js[b] - R * stride >= 0)
                def _(b=b):
                    for h in range(2):
                        pltpu.make_async_copy(
                            rows_v.at[b, pl.ds(h * (CH // 2), CH // 2)],
                            slab.at[idx_v.at[b, 2 + h]], ssem.at[b]).wait()

            for b in range(R):
                @pl.when(js[b] < NCH)
                def _(b=b):
                    pltpu.async_copy(ei_hbm.at[js[b]], idx_v.at[b], isem)

            for b in range(R):
                @pl.when(js[b] < NCH)
                def _(b=b):
                    pltpu.make_async_copy(ei_hbm.at[js[b]], idx_v.at[b],
                                          isem).wait()
                    for h in range(2):
                        pltpu.async_copy(
                            g_hbm.at[idx_v.at[b, h]],
                            rows_v.at[b, pl.ds(h * (CH // 2), CH // 2)],
                            gsem.at[b])

            for b in range(R):
                @pl.when(js[b] < NCH)
                def _(b=b):
                    for h in range(2):
                        sl = pl.ds(h * (CH // 2), CH // 2)
                        pltpu.make_async_copy(
                            g_hbm.at[idx_v.at[b, h]], rows_v.at[b, sl],
                            gsem.at[b]).wait()
                        pltpu.async_copy(rows_v.at[b, sl],
                                         slab.at[idx_v.at[b, 2 + h]],
                                         ssem.at[b], add=True)

            return carry

        nbody = (nloop + R - 1) // R
        lax.fori_loop(0, nbody, body, 0)
        for b in range(R):
            @pl.when((nbody - 1) * R * stride + b * stride + base < NCH)
            def _(b=b):
                for h in range(2):
                    pltpu.make_async_copy(
                        rows_v.at[b, pl.ds(h * (CH // 2), CH // 2)],
                        slab.at[idx_v.at[b, 2 + h]], ssem.at[b]).wait()
        plsc.subcore_barrier()
        pltpu.sync_copy(slab.at[pl.ds(s * RPB, RPB)],
                        out_hbm.at[c, pl.ds(s * RPB, RPB)])

        @pl.when(s == 0)
        def _drain_tail():
            pltpu.sync_copy(slab.at[pl.ds(NS * RPB, N - NS * RPB)],
                            out_hbm.at[c, pl.ds(NS * RPB, N - NS * RPB)])

    return k


_PROP_CACHE = {}


def _prop_edge(ei4, g):
    if 0 not in _PROP_CACHE:
        _PROP_CACHE[0] = _make_prop()
    return _PROP_CACHE[0](ei4, g)


_BR = 2000


def _dinv_call(partials, x):
    def body(p_ref, x_ref, d_ref, xs_ref):
        deg = jnp.sum(p_ref[...], axis=0) + 1.0
        d = lax.rsqrt(deg)[:, None]
        d_ref[...] = d
        xs_ref[...] = d * x_ref[...]

    return pl.pallas_call(
        body,
        out_shape=(jax.ShapeDtypeStruct((N, 1), jnp.float32),
                   jax.ShapeDtypeStruct((N, 128), jnp.float32)),
    )(partials, x)


def _lin12_call(S1, xs, dinv, W1, b1r, W2):
    def body(s_ref, x_ref, d_ref, w1_ref, b_ref, w2_ref, o_ref):
        d = d_ref[...]
        t = s_ref[0] + s_ref[1] + x_ref[...]
        h = jnp.dot(t, w1_ref[...], preferred_element_type=jnp.float32)
        u = jnp.maximum(d * h + b_ref[...], 0.0)
        o_ref[...] = d * jnp.dot(u, w2_ref[...],
                                 preferred_element_type=jnp.float32)

    return pl.pallas_call(
        body,
        grid=(N // _BR,),
        in_specs=[
            pl.BlockSpec((NC, _BR, F), lambda i: (0, i, 0)),
            pl.BlockSpec((_BR, F), lambda i: (i, 0)),
            pl.BlockSpec((_BR, 1), lambda i: (i, 0)),
            pl.BlockSpec((128, 256), lambda i: (0, 0)),
            pl.BlockSpec((1, 256), lambda i: (0, 0)),
            pl.BlockSpec((256, F), lambda i: (0, 0)),
        ],
        out_specs=pl.BlockSpec((_BR, F), lambda i: (i, 0)),
        out_shape=jax.ShapeDtypeStruct((N, F), jnp.float32),
    )(S1, xs, dinv, W1, b1r, W2)


def _lin_mid_call(S, g, dinv, br, W):
    def body(s_ref, g_ref, d_ref, b_ref, w_ref, o_ref):
        d = d_ref[...]
        u = jnp.maximum(d * (s_ref[0] + s_ref[1] + g_ref[...]) + b_ref[...],
                        0.0)
        o_ref[...] = d * jnp.dot(u, w_ref[...],
                                 preferred_element_type=jnp.float32)

    return pl.pallas_call(
        body,
        grid=(N // _BR,),
        in_specs=[
            pl.BlockSpec((NC, _BR, F), lambda i: (0, i, 0)),
            pl.BlockSpec((_BR, F), lambda i: (i, 0)),
            pl.BlockSpec((_BR, 1), lambda i: (i, 0)),
            pl.BlockSpec((1, F), lambda i: (0, 0)),
            pl.BlockSpec((F, F), lambda i: (0, 0)),
        ],
        out_specs=pl.BlockSpec((_BR, F), lambda i: (i, 0)),
        out_shape=jax.ShapeDtypeStruct((N, F), jnp.float32),
    )(S, g, dinv, br, W)


def _head_call(S, g, dinv, b5r, Wm1p, bm1p, Wm2p, bm2p):
    def body(s_ref, g_ref, d_ref, b_ref, w1_ref, c1_ref, w2_ref, c2_ref,
             o_ref):
        d = d_ref[...]
        u = jnp.maximum(d * (s_ref[0] + s_ref[1] + g_ref[...]) + b_ref[...],
                        0.0)
        h2 = jnp.maximum(
            jnp.dot(u, w1_ref[...], preferred_element_type=jnp.float32)
            + c1_ref[...], 0.0)
        o_ref[...] = jnp.dot(
            h2, w2_ref[...], preferred_element_type=jnp.float32) + c2_ref[...]

    return pl.pallas_call(
        body,
        grid=(N // _BR,),
        in_specs=[
            pl.BlockSpec((NC, _BR, F), lambda i: (0, i, 0)),
            pl.BlockSpec((_BR, F), lambda i: (i, 0)),
            pl.BlockSpec((_BR, 1), lambda i: (i, 0)),
            pl.BlockSpec((1, F), lambda i: (0, 0)),
            pl.BlockSpec((F, F), lambda i: (0, 0)),
            pl.BlockSpec((1, F), lambda i: (0, 0)),
            pl.BlockSpec((F, F), lambda i: (0, 0)),
            pl.BlockSpec((1, F), lambda i: (0, 0)),
        ],
        out_specs=pl.BlockSpec((_BR, F), lambda i: (i, 0)),
        out_shape=jax.ShapeDtypeStruct((N, F), jnp.float32),
    )(S, g, dinv, b5r, Wm1p, bm1p, Wm2p, bm2p)


def kernel(x, edge_index, batch, W1, b1, W2, b2, W3, b3, W4, b4, W5, b5,
           Wm1, bm1, Wm2, bm2):
    del batch
    src2d = edge_index[0].reshape(NCH, CH)
    dst2d = edge_index[1].reshape(NCH, CH)
    ei4 = jnp.concatenate(
        [edge_index[0].reshape(NCH, 2, CH // 2),
         edge_index[1].reshape(NCH, 2, CH // 2)], axis=1)

    partials = _degree_call(dst2d)
    dinv, xs = _dinv_call(partials, x)

    S1 = _prop_edge(ei4, xs)
    g = _lin12_call(S1, xs, dinv, W1, b1.reshape(1, 256), W2)
    for (bl, Wn) in ((b2, W3), (b3, W4), (b4, W5)):
        S = _prop_edge(ei4, g)
        g = _lin_mid_call(S, g, dinv, bl.reshape(1, F), Wn)

    S5 = _prop_edge(ei4, g)

    hm = Wm1.shape[1]
    Wm1p = jnp.pad(Wm1, ((0, 0), (0, F - hm)))
    bm1p = jnp.pad(bm1, (0, F - hm)).reshape(1, F)
    Wm2p = jnp.pad(Wm2, ((0, F - hm), (0, F - Wm2.shape[1])))
    bm2p = jnp.pad(bm2, (0, F - Wm2.shape[1])).reshape(1, F)

    out = _head_call(S5, g, dinv, b5.reshape(1, F), Wm1p, bm1p, Wm2p, bm2p)
    return out[:, :Wm2.shape[1]]
